# baseline (speedup 1.0000x reference)
import functools

import jax
import jax.numpy as jnp
from jax import lax
from jax.experimental import pallas as pl
from jax.experimental.pallas import tpu as pltpu
from jax.experimental.pallas import tpu_sc as plsc

N = 10000
E = 320000
D = 128
DP = D // 2
LANES = 16
NC = 2
NS = 16
NW = NC * NS
EPW = E // NW
C = 80
NCHUNK = EPW // C
NBUF = 3
NG = (NCHUNK + NBUF - 1) // NBUF
NPAD = 10240
RPT = NPAD // NS
ZROWS = 16

_mesh = plsc.VectorSubcoreMesh(core_axis_name="c", subcore_axis_name="s")


@functools.partial(
    pl.kernel,
    out_type=jax.ShapeDtypeStruct((NC, NPAD, D), jnp.float32),
    mesh=_mesh,
    compiler_params=pltpu.CompilerParams(use_tc_tiling_on_sc=False),
    scratch_types=[
        pltpu.VMEM((NBUF, C), jnp.int32),
        pltpu.VMEM((NBUF, C), jnp.int32),
        pltpu.VMEM((NBUF, C, DP), jnp.int32),
        pltpu.VMEM((NBUF, C, D), jnp.float32),
        pltpu.VMEM((ZROWS, D), jnp.float32),
        pltpu.VMEM_SHARED((NPAD, D), jnp.float32),
        pltpu.SemaphoreType.DMA,
        pltpu.SemaphoreType.DMA,
        pltpu.SemaphoreType.DMA,
        pltpu.SemaphoreType.DMA,
        pltpu.SemaphoreType.DMA,
        pltpu.SemaphoreType.DMA,
        pltpu.SemaphoreType.DMA,
        pltpu.SemaphoreType.DMA,
        pltpu.SemaphoreType.DMA,
        pltpu.SemaphoreType.DMA,
        pltpu.SemaphoreType.DMA,
        pltpu.SemaphoreType.DMA,
        pltpu.SemaphoreType.DMA,
        pltpu.SemaphoreType.DMA,
        pltpu.SemaphoreType.DMA,
        pltpu.SemaphoreType.DMA,
    ],
)
def _sc_aggregate(xp_hbm, src_hbm, dst_hbm, ea_hbm, out_hbm,
                  sidxr, didxb, rows, ea, zbuf, acc,
                  g0, g1, g2, e0, e1, e2, s0, s1, s2,
                  x0, x1, x2, d0, d1, d2, zsem):
    c = lax.axis_index("c")
    s = lax.axis_index("s")
    gsem = (g0, g1, g2)
    esem = (e0, e1, e2)
    ssem = (s0, s1, s2)
    xsem = (x0, x1, x2)
    dsem = (d0, d1, d2)
    wid = s * NC + c
    ebase = wid * EPW

    def _issue_sidx(i, j):
        pltpu.async_copy(src_hbm.at[pl.ds(ebase + i * C, C)], sidxr.at[j],
                         xsem[j])

    def _issue_didx(i, j):
        pltpu.async_copy(dst_hbm.at[pl.ds(ebase + i * C, C)], didxb.at[j],
                         dsem[j])

    def _wait_sidx(j):
        pltpu.make_async_copy(src_hbm.at[pl.ds(0, C)], sidxr.at[j],
                              xsem[j]).wait()

    def _issue_data(i, b):
        pltpu.async_copy(xp_hbm.at[sidxr.at[b]], rows.at[b], gsem[b])
        pltpu.async_copy(ea_hbm.at[pl.ds(ebase + i * C, C)], ea.at[b],
                         esem[b])

    def _drain_scatter(b):
        pltpu.make_async_copy(ea_hbm.at[pl.ds(0, C)], ea.at[b],
                              ssem[b]).wait()

    for j in range(NBUF):
        _issue_sidx(j, j)
        _issue_didx(j, j)

    zero = jnp.zeros((LANES,), jnp.float32)

    @plsc.parallel_loop(0, ZROWS)
    def _zrow(i):
        for j in range(D // LANES):
            zbuf[i, pl.ds(j * LANES, LANES)] = zero

    base_r = s * RPT
    zcps = [
        pltpu.async_copy(zbuf, acc.at[pl.ds(base_r + k * ZROWS, ZROWS)], zsem)
        for k in range(RPT // ZROWS)
    ]
    _wait_sidx(0)
    _issue_data(0, 0)
    _wait_sidx(1)
    _issue_data(1, 1)
    for cp in zcps:
        cp.wait()
    plsc.subcore_barrier()

    def _group(g, carry):
        for b in range(NBUF):
            v = g * NBUF + b
            bp = (b + 2) % NBUF

            @pl.when(jnp.logical_and(v >= 1, v <= NCHUNK - 2))
            def _():
                _drain_scatter(bp)

            @pl.when(jnp.logical_and(v >= 1, v + 2 < NCHUNK))
            def _():
                _issue_didx(v + 2, bp)

            @pl.when(v + 2 < NCHUNK)
            def _():
                _wait_sidx(bp)
                _issue_data(v + 2, bp)

            @pl.when(v < NCHUNK)
            def _():
                pltpu.make_async_copy(xp_hbm.at[pl.ds(0, C)], rows.at[b],
                                      gsem[b]).wait()

                @pl.when(v + 3 < NCHUNK)
                def _():
                    _issue_sidx(v + 3, b)

                pltpu.make_async_copy(ea_hbm.at[pl.ds(0, C)], ea.at[b],
                                      esem[b]).wait()
                rows_b = rows.at[b]
                ea_b = ea.at[b]

                shift16 = jnp.full((LANES,), 16, jnp.int32)
                mask16 = jnp.full((LANES,), -65536, jnp.int32)

                @plsc.parallel_loop(0, C)
                def _msg_row(r):
                    for g2 in range(D // 32):
                        w = rows_b[r, pl.ds(g2 * LANES, LANES)]
                        lo = lax.bitcast_convert_type(lax.shift_left(w, shift16), jnp.float32)
                        hi = lax.bitcast_convert_type(jnp.bitwise_and(w, mask16), jnp.float32)
                        sl = pl.ds(g2 * 32, LANES)
                        sh = pl.ds(g2 * 32 + LANES, LANES)
                        ea_b[r, sl] = jnp.maximum(lo + ea_b[r, sl], 0.0)
                        ea_b[r, sh] = jnp.maximum(hi + ea_b[r, sh], 0.0)

                pltpu.make_async_copy(dst_hbm.at[pl.ds(0, C)], didxb.at[b],
                                      dsem[b]).wait()
                pltpu.async_copy(ea.at[b], acc.at[didxb.at[b]],
                                 ssem[b], add=True)
        return carry

    lax.fori_loop(0, NG, _group, 0)
    for k in (NCHUNK - 2, NCHUNK - 1):
        _drain_scatter(k % NBUF)
    plsc.subcore_barrier()

    pltpu.async_copy(acc.at[pl.ds(base_r, RPT)],
                     out_hbm.at[c, pl.ds(base_r, RPT)], zsem)
    pltpu.make_async_copy(acc.at[pl.ds(base_r, RPT)],
                          out_hbm.at[c, pl.ds(base_r, RPT)], zsem).wait()


def _pack_rows(h):
    b32 = lax.bitcast_convert_type(h, jnp.int32)
    rnd = b32 + 0x7FFF + jnp.bitwise_and(lax.shift_right_logical(b32, 16), 1)
    bits = jnp.bitwise_and(lax.shift_right_logical(rnd, 16), 0xFFFF)
    words = [
        jnp.bitwise_or(bits[:, 32 * g:32 * g + LANES],
                       lax.shift_left(bits[:, 32 * g + LANES:32 * (g + 1)],
                                      16))
        for g in range(D // 32)
    ]
    return jnp.concatenate(words, axis=1)


def _pack_body(x_ref, o_ref):
    o_ref[...] = _pack_rows(x_ref[...])


_pack = pl.pallas_call(
    _pack_body,
    out_shape=jax.ShapeDtypeStruct((N, DP), jnp.int32),
)


def _mlp_body(x_ref, p_ref, w1_ref, b1_ref, g_ref, be_ref, w2_ref, b2_ref,
              o_ref, op_ref):
    h = x_ref[...] + p_ref[0, :N] + p_ref[1, :N]
    t = jnp.dot(h, w1_ref[...], preferred_element_type=jnp.float32)
    t = t + b1_ref[...]
    mean = jnp.mean(t, axis=0, keepdims=True)
    var = jnp.mean((t - mean) * (t - mean), axis=0, keepdims=True)
    t = (t - mean) * lax.rsqrt(var + 1e-5) * g_ref[...] + be_ref[...]
    t = jnp.maximum(t, 0.0)
    t = jnp.dot(t, w2_ref[...], preferred_element_type=jnp.float32)
    t = t + b2_ref[...]
    t = jnp.maximum(t, 0.0)
    o_ref[...] = t
    op_ref[...] = _pack_rows(t)


_mlp = pl.pallas_call(
    _mlp_body,
    out_shape=(
        jax.ShapeDtypeStruct((N, D), jnp.float32),
        jax.ShapeDtypeStruct((N, DP), jnp.int32),
    ),
)


def kernel(x, edge_index, edge_attr,
           W1_0, b1_0, gamma_0, beta_0, W2_0, b2_0,
           W1_1, b1_1, gamma_1, beta_1, W2_1, b2_1):
    src = edge_index[0]
    dst = edge_index[1]
    params = [
        (W1_0, b1_0, gamma_0, beta_0, W2_0, b2_0),
        (W1_1, b1_1, gamma_1, beta_1, W2_1, b2_1),
    ]
    h = x
    hp = _pack(x)
    for (W1, b1, gamma, beta, W2, b2) in params:
        partials = _sc_aggregate(hp, src, dst, edge_attr)
        h, hp = _mlp(h, partials,
                     W1, b1.reshape(1, D), gamma.reshape(1, D),
                     beta.reshape(1, D), W2, b2.reshape(1, D))
    return h


# speedup vs baseline: 1.0394x; 1.0394x over previous
import functools

import jax
import jax.numpy as jnp
from jax import lax
from jax.experimental import pallas as pl
from jax.experimental.pallas import tpu as pltpu
from jax.experimental.pallas import tpu_sc as plsc

N = 10000
E = 320000
D = 128
DP = D // 2
LANES = 16
NC = 2
NS = 16
NW = NC * NS
EPW = E // NW
C = 40
NCHUNK = EPW // C
NBUF = 4
NIB = 4
NG = (NCHUNK + NBUF - 1) // NBUF
NPAD = 10240
RPT = NPAD // NS
ZROWS = 32

_mesh = plsc.VectorSubcoreMesh(core_axis_name="c", subcore_axis_name="s")


@functools.partial(
    pl.kernel,
    out_type=jax.ShapeDtypeStruct((NC, NPAD, D), jnp.float32),
    mesh=_mesh,
    compiler_params=pltpu.CompilerParams(use_tc_tiling_on_sc=False),
    scratch_types=[
        pltpu.VMEM((EPW,), jnp.int32),
        pltpu.VMEM((NIB, C), jnp.int32),
        pltpu.VMEM((NBUF, C, DP), jnp.int32),
        pltpu.VMEM((NBUF, C, D), jnp.float32),
        pltpu.VMEM((ZROWS, D), jnp.float32),
        pltpu.VMEM_SHARED((NPAD, D), jnp.float32),
        pltpu.SemaphoreType.DMA,
        pltpu.SemaphoreType.DMA,
        pltpu.SemaphoreType.DMA,
        pltpu.SemaphoreType.DMA,
        pltpu.SemaphoreType.DMA,
        pltpu.SemaphoreType.DMA,
        pltpu.SemaphoreType.DMA,
        pltpu.SemaphoreType.DMA,
        pltpu.SemaphoreType.DMA,
        pltpu.SemaphoreType.DMA,
        pltpu.SemaphoreType.DMA,
        pltpu.SemaphoreType.DMA,
        pltpu.SemaphoreType.DMA,
        pltpu.SemaphoreType.DMA,
        pltpu.SemaphoreType.DMA,
        pltpu.SemaphoreType.DMA,
        pltpu.SemaphoreType.DMA,
        pltpu.SemaphoreType.DMA,
        pltpu.SemaphoreType.DMA,
        pltpu.SemaphoreType.DMA,
        pltpu.SemaphoreType.DMA,
    ],
)
def _sc_aggregate(xp_hbm, src_hbm, dst_hbm, ea_hbm, out_hbm,
                  sidx, didxb, rows, ea, zbuf, acc,
                  g0, g1, g2, g3, e0, e1, e2, e3, s0, s1, s2, s3, d0, d1, d2, d3,
                  h0, h1, h2, h3, zsem):
    c = lax.axis_index("c")
    s = lax.axis_index("s")
    gsem = (g0, g1, g2, g3)
    hsem = (h0, h1, h2, h3)
    esem = (e0, e1, e2, e3)
    ssem = (s0, s1, s2, s3)
    dsem = (d0, d1, d2, d3)
    wid = s * NC + c
    ebase = wid * EPW

    icp = pltpu.async_copy(src_hbm.at[pl.ds(ebase, EPW)], sidx, g0)
    zero = jnp.zeros((LANES,), jnp.float32)

    @plsc.parallel_loop(0, ZROWS)
    def _zrow(i):
        for j in range(D // LANES):
            zbuf[i, pl.ds(j * LANES, LANES)] = zero

    base_r = s * RPT
    zcps = [
        pltpu.async_copy(zbuf, acc.at[pl.ds(base_r + k * ZROWS, ZROWS)], zsem)
        for k in range(RPT // ZROWS)
    ]
    icp.wait()

    def _issue_didx(i, j):
        pltpu.async_copy(dst_hbm.at[pl.ds(ebase + i * C, C)], didxb.at[j],
                         dsem[j])

    CH = 24

    def _issue_data(i, b):
        pltpu.async_copy(xp_hbm.at[sidx.at[pl.ds(i * C, CH)]],
                         rows.at[b, pl.ds(0, CH)], gsem[b])
        pltpu.async_copy(xp_hbm.at[sidx.at[pl.ds(i * C + CH, C - CH)]],
                         rows.at[b, pl.ds(CH, C - CH)], hsem[b])
        pltpu.async_copy(ea_hbm.at[pl.ds(ebase + i * C, C)], ea.at[b],
                         esem[b])

    def _drain_scatter(b):
        pltpu.make_async_copy(ea_hbm.at[pl.ds(0, C)], ea.at[b],
                              ssem[b]).wait()

    _issue_didx(0, 0)
    _issue_didx(1, 1)
    _issue_data(0, 0)
    _issue_data(1, 1)
    for cp in zcps:
        cp.wait()
    plsc.subcore_barrier()

    def _group(g, carry):
        for b in range(NBUF):
            v = g * NBUF + b
            bp = (b + 2) % NBUF

            @pl.when(jnp.logical_and(v >= 2, v + 2 < NCHUNK))
            def _():
                _drain_scatter(bp)

            @pl.when(v + 2 < NCHUNK)
            def _():
                for j in range(NIB):
                    @pl.when((v + 2) % NIB == j)
                    def _():
                        _issue_didx(v + 2, j)
                _issue_data(v + 2, bp)

            @pl.when(v < NCHUNK)
            def _():
                pltpu.make_async_copy(xp_hbm.at[pl.ds(0, CH)],
                                      rows.at[b, pl.ds(0, CH)],
                                      gsem[b]).wait()
                pltpu.make_async_copy(xp_hbm.at[pl.ds(0, C - CH)],
                                      rows.at[b, pl.ds(CH, C - CH)],
                                      hsem[b]).wait()
                pltpu.make_async_copy(ea_hbm.at[pl.ds(0, C)], ea.at[b],
                                      esem[b]).wait()
                rows_b = rows.at[b]
                ea_b = ea.at[b]

                shift16 = jnp.full((LANES,), 16, jnp.int32)
                mask16 = jnp.full((LANES,), -65536, jnp.int32)

                @plsc.parallel_loop(0, C)
                def _msg_row(r):
                    for g2 in range(D // 32):
                        w = rows_b[r, pl.ds(g2 * LANES, LANES)]
                        lo = lax.bitcast_convert_type(lax.shift_left(w, shift16), jnp.float32)
                        hi = lax.bitcast_convert_type(jnp.bitwise_and(w, mask16), jnp.float32)
                        sl = pl.ds(g2 * 32, LANES)
                        sh = pl.ds(g2 * 32 + LANES, LANES)
                        ea_b[r, sl] = jnp.maximum(lo + ea_b[r, sl], 0.0)
                        ea_b[r, sh] = jnp.maximum(hi + ea_b[r, sh], 0.0)

                for j in range(NIB):
                    @pl.when(v % NIB == j)
                    def _():
                        pltpu.make_async_copy(
                            dst_hbm.at[pl.ds(0, C)], didxb.at[j],
                            dsem[j]).wait()
                        pltpu.async_copy(ea.at[b], acc.at[didxb.at[j]],
                                         ssem[b], add=True)
        return carry

    lax.fori_loop(0, NG, _group, 0)
    for b in range(NBUF):
        _drain_scatter(b)
    plsc.subcore_barrier()

    wcps = [
        pltpu.async_copy(acc.at[pl.ds(base_r + k * ZROWS, ZROWS)],
                         out_hbm.at[c, pl.ds(base_r + k * ZROWS, ZROWS)],
                         zsem)
        for k in range(RPT // ZROWS)
    ]
    for cp in wcps:
        cp.wait()


def _pack_rows(h):
    b32 = lax.bitcast_convert_type(h, jnp.int32)
    rnd = b32 + 0x7FFF + jnp.bitwise_and(lax.shift_right_logical(b32, 16), 1)
    bits = jnp.bitwise_and(lax.shift_right_logical(rnd, 16), 0xFFFF)
    words = [
        jnp.bitwise_or(bits[:, 32 * g:32 * g + LANES],
                       lax.shift_left(bits[:, 32 * g + LANES:32 * (g + 1)],
                                      16))
        for g in range(D // 32)
    ]
    return jnp.concatenate(words, axis=1)


def _pack_body(x_ref, o_ref):
    o_ref[...] = _pack_rows(x_ref[...])


_pack = pl.pallas_call(
    _pack_body,
    out_shape=jax.ShapeDtypeStruct((N, DP), jnp.int32),
)


def _mlp_body(x_ref, p_ref, w1_ref, b1_ref, g_ref, be_ref, w2_ref, b2_ref,
              o_ref, op_ref):
    h = x_ref[...] + p_ref[0, :N] + p_ref[1, :N]
    t = jnp.dot(h, w1_ref[...], preferred_element_type=jnp.float32)
    t = t + b1_ref[...]
    mean = jnp.mean(t, axis=0, keepdims=True)
    var = jnp.mean((t - mean) * (t - mean), axis=0, keepdims=True)
    t = (t - mean) * lax.rsqrt(var + 1e-5) * g_ref[...] + be_ref[...]
    t = jnp.maximum(t, 0.0)
    t = jnp.dot(t, w2_ref[...], preferred_element_type=jnp.float32)
    t = t + b2_ref[...]
    t = jnp.maximum(t, 0.0)
    o_ref[...] = t
    op_ref[...] = _pack_rows(t)


_mlp = pl.pallas_call(
    _mlp_body,
    out_shape=(
        jax.ShapeDtypeStruct((N, D), jnp.float32),
        jax.ShapeDtypeStruct((N, DP), jnp.int32),
    ),
)


def kernel(x, edge_index, edge_attr,
           W1_0, b1_0, gamma_0, beta_0, W2_0, b2_0,
           W1_1, b1_1, gamma_1, beta_1, W2_1, b2_1):
    src = edge_index[0]
    dst = edge_index[1]
    params = [
        (W1_0, b1_0, gamma_0, beta_0, W2_0, b2_0),
        (W1_1, b1_1, gamma_1, beta_1, W2_1, b2_1),
    ]
    h = x
    hp = _pack(x)
    for (W1, b1, gamma, beta, W2, b2) in params:
        partials = _sc_aggregate(hp, src, dst, edge_attr)
        h, hp = _mlp(h, partials,
                     W1, b1.reshape(1, D), gamma.reshape(1, D),
                     beta.reshape(1, D), W2, b2.reshape(1, D))
    return h
